# baseline (speedup 1.0000x reference)
import functools
import jax
import jax.numpy as jnp
from jax import lax
from jax.experimental import pallas as pl
from jax.experimental.pallas import tpu as pltpu
from jax.experimental.pallas import tpu_sc as plsc


def kernel(x, timestamps, pe, hour_emb, day_emb, month_emb, season_emb):
    B, L, D = x.shape
    N = B * L
    T = 32
    NW = 32
    per_w = N // NW
    n_it = per_w // T
    GROUPS = T * D // 16

    xf = x.reshape(N, D)
    pe2 = pe[0]

    i0 = jnp.arange(256, dtype=jnp.int32)
    combo = jnp.concatenate([
        hour_emb[i0 % 4], day_emb[(i0 // 4) % 4],
        month_emb[(i0 // 16) % 4], season_emb[(i0 // 64) % 4]], axis=1)

    ts = timestamps.reshape(N, 4)
    cidx = (ts[:, 0] + 4 * ts[:, 1] + 16 * ts[:, 2] + 64 * ts[:, 3]).astype(jnp.int32)
    peidx = (jnp.arange(N, dtype=jnp.int32) % L).astype(jnp.int32)

    mesh = plsc.VectorSubcoreMesh(core_axis_name="c", subcore_axis_name="s")

    @functools.partial(
        pl.kernel, mesh=mesh,
        out_type=jax.ShapeDtypeStruct((N, D), jnp.float32),
        scratch_types=[
            pltpu.VMEM((T, D), jnp.float32),
            pltpu.VMEM((T, D), jnp.float32),
            pltpu.VMEM((T, D), jnp.float32),
            pltpu.VMEM((T,), jnp.int32),
            pltpu.VMEM((T,), jnp.int32),
            pltpu.SemaphoreType.DMA,
            pltpu.SemaphoreType.DMA,
            pltpu.SemaphoreType.DMA,
        ],
    )
    def k(xf_hbm, pe_hbm, combo_hbm, cidx_hbm, peidx_hbm, out_hbm,
          bufx, bufp, bufc, pei, ci, sem1, sem2, sem3):
        wid = lax.axis_index("s") * 2 + lax.axis_index("c")
        gpt = D // 16

        def body(it, _):
            base = wid * per_w + it * T
            pltpu.sync_copy(cidx_hbm.at[pl.ds(base, T)], ci)
            pltpu.sync_copy(peidx_hbm.at[pl.ds(base, T)], pei)
            cpx = pltpu.async_copy(xf_hbm.at[pl.ds(base, T)], bufx, sem1)
            cpp = pltpu.async_copy(pe_hbm.at[pei], bufp, sem2)
            cpc = pltpu.async_copy(combo_hbm.at[ci], bufc, sem3)
            cpx.wait()
            cpp.wait()
            cpc.wait()

            @plsc.parallel_loop(0, GROUPS, 1, unroll=8)
            def add_body(g):
                t = g // gpt
                sl = pl.ds((g % gpt) * 16, 16)
                bufx[t, sl] = bufx[t, sl] + bufp[t, sl] + bufc[t, sl]
            pltpu.sync_copy(bufx, out_hbm.at[pl.ds(base, T)])
            return 0

        lax.fori_loop(0, n_it, body, 0)

    out = k(xf, pe2, combo, cidx, peidx)
    return out.reshape(B, L, D)


# speedup vs baseline: 3.8893x; 3.8893x over previous
import jax
import jax.numpy as jnp
from jax import lax
from jax.experimental import pallas as pl


def _body(ts_ref, x_ref, pe_ref, h_ref, d_ref, m_ref, s_ref, out_ref):
    S = x_ref.shape[1]
    D = x_ref.shape[2]
    C = D // 4
    xb = x_ref[0]
    peb = pe_ref[0]
    for c, emb in enumerate((h_ref, d_ref, m_ref, s_ref)):
        idx = ts_ref[0, :, c]
        R = emb.shape[0]
        oh = (idx[:, None] == lax.broadcasted_iota(jnp.int32, (S, R), 1))
        chunk = jnp.dot(oh.astype(jnp.float32), emb[...],
                        preferred_element_type=jnp.float32)
        out_ref[0, :, c * C:(c + 1) * C] = (
            xb[:, c * C:(c + 1) * C] + peb[:, c * C:(c + 1) * C] + chunk)


def kernel(x, timestamps, pe, hour_emb, day_emb, month_emb, season_emb):
    B, L, D = x.shape
    S = 2048
    nsb = L // S

    def full(a):
        return pl.BlockSpec(a.shape, lambda i, j: (0,) * a.ndim)

    return pl.pallas_call(
        _body,
        grid=(nsb, B),
        in_specs=[
            pl.BlockSpec((1, S, 4), lambda i, j: (j, i, 0)),
            pl.BlockSpec((1, S, D), lambda i, j: (j, i, 0)),
            pl.BlockSpec((1, S, D), lambda i, j: (0, i, 0)),
            full(hour_emb), full(day_emb), full(month_emb), full(season_emb),
        ],
        out_specs=pl.BlockSpec((1, S, D), lambda i, j: (j, i, 0)),
        out_shape=jax.ShapeDtypeStruct((B, L, D), x.dtype),
    )(timestamps, x, pe, hour_emb, day_emb, month_emb, season_emb)


# speedup vs baseline: 4.0327x; 1.0369x over previous
import jax
import jax.numpy as jnp
from jax import lax
from jax.experimental import pallas as pl


def _body(ts_ref, x_ref, pe_ref, emb_ref, out_ref):
    S = x_ref.shape[1]
    W = x_ref.shape[2]
    C = 256
    d = pl.program_id(0)
    ts = ts_ref[0]
    xb = x_ref[0]
    peb = pe_ref[...]
    for c in range(2):
        idx = jnp.where(d == 0, ts[c, :], ts[2 + c, :])
        oh = (idx[:, None] == lax.broadcasted_iota(jnp.int32, (S, 32), 1))
        chunk = jnp.dot(oh.astype(jnp.float32),
                        emb_ref[:, c * C:(c + 1) * C],
                        preferred_element_type=jnp.float32)
        out_ref[0, :, c * C:(c + 1) * C] = (
            xb[:, c * C:(c + 1) * C] + peb[:, c * C:(c + 1) * C] + chunk)


def kernel(x, timestamps, pe, hour_emb, day_emb, month_emb, season_emb):
    B, L, D = x.shape
    W = D // 2

    pe2 = pe[0]
    tsT = timestamps.transpose(0, 2, 1)

    def pad32(e):
        return jnp.pad(e, ((0, 32 - e.shape[0]), (0, 0)))

    emb = jnp.concatenate(
        [pad32(hour_emb), pad32(day_emb), pad32(month_emb), pad32(season_emb)],
        axis=1)

    return pl.pallas_call(
        _body,
        grid=(2, B),
        in_specs=[
            pl.BlockSpec((1, 4, L), lambda i, j: (j, 0, 0)),
            pl.BlockSpec((1, L, W), lambda i, j: (j, 0, i)),
            pl.BlockSpec((L, W), lambda i, j: (0, i)),
            pl.BlockSpec((32, W), lambda i, j: (0, i)),
        ],
        out_specs=pl.BlockSpec((1, L, W), lambda i, j: (j, 0, i)),
        out_shape=jax.ShapeDtypeStruct((B, L, D), x.dtype),
    )(tsT, x, pe2, emb)


# speedup vs baseline: 4.3592x; 1.0810x over previous
import jax
import jax.numpy as jnp
from jax import lax
from jax.experimental import pallas as pl
from jax.experimental.pallas import tpu as pltpu

_RING = 4
_RCH = 512


def _body(ts_ref, emb_ref, x_hbm, pe_hbm, out_hbm,
          pe_buf, x_buf, out_buf, sem_pe, sem_x, sem_out):
    B = x_hbm.shape[0]
    L = x_hbm.shape[1]
    D = x_hbm.shape[2]
    C = D // 4
    NR = L // _RCH
    NU = NR * B

    def x_copy(u, slot):
        r, b = divmod(u, B)
        return pltpu.make_async_copy(
            x_hbm.at[b, pl.ds(r * _RCH, _RCH), :], x_buf.at[slot],
            sem_x.at[slot])

    def pe_copy(r):
        return pltpu.make_async_copy(
            pe_hbm.at[pl.ds(r * _RCH, _RCH), :], pe_buf.at[r], sem_pe.at[r])

    def out_copy(u, slot):
        r, b = divmod(u, B)
        return pltpu.make_async_copy(
            out_buf.at[slot], out_hbm.at[b, pl.ds(r * _RCH, _RCH), :],
            sem_out.at[slot])

    pe_copy(0).start()
    for k in range(_RING):
        x_copy(k, k).start()
    for r in range(1, NR):
        pe_copy(r).start()

    for u in range(NU):
        r, b = divmod(u, B)
        slot = u % _RING
        x_copy(u, slot).wait()
        if b == 0:
            pe_copy(r).wait()
        if u >= _RING:
            out_copy(u - _RING, slot).wait()

        xb = x_buf[slot]
        peb = pe_buf[r]
        for c in range(4):
            idx = ts_ref[b, c, pl.ds(r * _RCH, _RCH)]
            oh = (idx[:, None] ==
                  lax.broadcasted_iota(jnp.int32, (_RCH, 32), 1))
            chunk = jnp.dot(oh.astype(jnp.float32),
                            emb_ref[:, c * C:(c + 1) * C],
                            preferred_element_type=jnp.float32)
            out_buf[slot, :, c * C:(c + 1) * C] = (
                xb[:, c * C:(c + 1) * C] + peb[:, c * C:(c + 1) * C] + chunk)

        out_copy(u, slot).start()
        if u + _RING < NU:
            x_copy(u + _RING, slot).start()

    for k in range(_RING):
        out_copy(NU - _RING + k, (NU - _RING + k) % _RING).wait()


def kernel(x, timestamps, pe, hour_emb, day_emb, month_emb, season_emb):
    B, L, D = x.shape
    NR = L // _RCH

    pe2 = pe[0]
    tsT = timestamps.transpose(0, 2, 1)

    def pad32(e):
        return jnp.pad(e, ((0, 32 - e.shape[0]), (0, 0)))

    emb = jnp.concatenate(
        [pad32(hour_emb), pad32(day_emb), pad32(month_emb), pad32(season_emb)],
        axis=1)

    return pl.pallas_call(
        _body,
        in_specs=[
            pl.BlockSpec((B, 4, L), lambda: (0, 0, 0)),
            pl.BlockSpec((32, D), lambda: (0, 0)),
            pl.BlockSpec(memory_space=pl.ANY),
            pl.BlockSpec(memory_space=pl.ANY),
        ],
        out_specs=pl.BlockSpec(memory_space=pl.ANY),
        out_shape=jax.ShapeDtypeStruct((B, L, D), x.dtype),
        scratch_shapes=[
            pltpu.VMEM((NR, _RCH, D), jnp.float32),
            pltpu.VMEM((_RING, _RCH, D), jnp.float32),
            pltpu.VMEM((_RING, _RCH, D), jnp.float32),
            pltpu.SemaphoreType.DMA((NR,)),
            pltpu.SemaphoreType.DMA((_RING,)),
            pltpu.SemaphoreType.DMA((_RING,)),
        ],
    )(tsT, emb, x, pe2)
